# Initial kernel scaffold; baseline (speedup 1.0000x reference)
import jax
import jax.numpy as jnp
from jax.experimental import pallas as pl


def kernel(node_type, edge_index, distance, node_index, source_index, target_index, select_edge_index, embedding, edge_mask, conv_params, W_nt1, b_nt1, W_nt2, b_nt2, W_et1, b_et1, W_et2, b_et2):
    raise NotImplementedError("write your pallas kernel here")


# speedup vs baseline: 3.1599x; 3.1599x over previous
import functools

import jax
import jax.numpy as jnp
from jax import lax
from jax.experimental import pallas as pl
from jax.experimental.pallas import tpu as pltpu
from jax.experimental.pallas import tpu_sc as plsc

DIM = 64
CUTOFF = 5.0
N_CENTERS = 50
L = 16
NC = 2
NS = 16
NW = NC * NS

_INTERPRET = False


def _softplus(x, beta=0.5, threshold=14.0):
    return jnp.where(beta * x > threshold, x,
                     (1.0 / beta) * jnp.log1p(jnp.exp(jnp.minimum(beta * x, threshold))))


def _tc_h(dist_m, edge_mask, w1, b1, w2, b2, tile=1024):
    epad = dist_m.shape[0]
    gap = CUTOFF / (N_CENTERS - 1)

    def body(d_ref, em_ref, w1_ref, b1_ref, w2_ref, b2_ref, o_ref):
        d2 = d_ref[...][:, None]
        centers = lax.broadcasted_iota(
            jnp.int32, (1, N_CENTERS), 1).astype(jnp.float32) * gap
        rbf = jnp.exp((-1.0 / gap) * (d2 - centers) ** 2)
        rbf = jnp.where(d2 < 0.0, em_ref[...][None, :], rbf)
        hh = _softplus(jnp.dot(rbf, w1_ref[...], preferred_element_type=jnp.float32)
                       + b1_ref[...][None, :])
        o_ref[...] = (jnp.dot(hh, w2_ref[...], preferred_element_type=jnp.float32)
                      + b2_ref[...][None, :])

    return pl.pallas_call(
        body,
        grid=(epad // tile,),
        in_specs=[
            pl.BlockSpec((tile,), lambda i: (i,)),
            pl.BlockSpec((N_CENTERS,), lambda i: (0,)),
            pl.BlockSpec((N_CENTERS, DIM), lambda i: (0, 0)),
            pl.BlockSpec((DIM,), lambda i: (0,)),
            pl.BlockSpec((DIM, DIM), lambda i: (0, 0)),
            pl.BlockSpec((DIM,), lambda i: (0,)),
        ],
        out_specs=pl.BlockSpec((tile, DIM), lambda i: (i, 0)),
        out_shape=jax.ShapeDtypeStruct((epad, DIM), jnp.float32),
        interpret=_INTERPRET,
    )(dist_m, edge_mask, w1, b1, w2, b2)


def _tc_matmul(x, w, tile=1024):
    n = x.shape[0]

    def body(x_ref, w_ref, o_ref):
        o_ref[...] = jnp.dot(x_ref[...], w_ref[...], preferred_element_type=jnp.float32)

    return pl.pallas_call(
        body,
        grid=(n // tile,),
        in_specs=[pl.BlockSpec((tile, DIM), lambda i: (i, 0)),
                  pl.BlockSpec((DIM, DIM), lambda i: (0, 0))],
        out_specs=pl.BlockSpec((tile, DIM), lambda i: (i, 0)),
        out_shape=jax.ShapeDtypeStruct((n, DIM), jnp.float32),
        interpret=_INTERPRET,
    )(x, w)


def _tc_update(node, agg, w2, b2, w3, b3, tile=1024):
    n = node.shape[0]

    def body(n_ref, a_ref, w2_ref, b2_ref, w3_ref, b3_ref, o_ref):
        cf = _softplus(jnp.dot(a_ref[...], w2_ref[...], preferred_element_type=jnp.float32)
                       + b2_ref[...][None, :])
        o_ref[...] = n_ref[...] + (
            jnp.dot(cf, w3_ref[...], preferred_element_type=jnp.float32)
            + b3_ref[...][None, :])

    return pl.pallas_call(
        body,
        grid=(n // tile,),
        in_specs=[pl.BlockSpec((tile, DIM), lambda i: (i, 0)),
                  pl.BlockSpec((tile, DIM), lambda i: (i, 0)),
                  pl.BlockSpec((DIM, DIM), lambda i: (0, 0)),
                  pl.BlockSpec((DIM,), lambda i: (0,)),
                  pl.BlockSpec((DIM, DIM), lambda i: (0, 0)),
                  pl.BlockSpec((DIM,), lambda i: (0,))],
        out_specs=pl.BlockSpec((tile, DIM), lambda i: (i, 0)),
        out_shape=jax.ShapeDtypeStruct((n, DIM), jnp.float32),
        interpret=_INTERPRET,
    )(node, agg, w2, b2, w3, b3)


def _tc_heads(nsel, ssel, tsel, wn1, bn1, wn2p, bn2p, we1s, we1t, be1, we2p, be2p,
              tile=512):
    n = nsel.shape[0]

    def body(ns_ref, ss_ref, ts_ref, wn1_ref, bn1_ref, wn2_ref, bn2_ref,
             we1s_ref, we1t_ref, be1_ref, we2_ref, be2_ref, nt_ref, et_ref):
        f32 = jnp.float32
        t1 = jnp.dot(ns_ref[...], wn1_ref[...], preferred_element_type=f32) + bn1_ref[...][None, :]
        nt_ref[...] = jnp.dot(t1, wn2_ref[...], preferred_element_type=f32) + bn2_ref[...][None, :]
        e1 = (jnp.dot(ss_ref[...], we1s_ref[...], preferred_element_type=f32)
              + jnp.dot(ts_ref[...], we1t_ref[...], preferred_element_type=f32)
              + be1_ref[...][None, :])
        et_ref[...] = jnp.dot(e1, we2_ref[...], preferred_element_type=f32) + be2_ref[...][None, :]

    full = lambda *shape: pl.BlockSpec(shape, lambda i: tuple(0 for _ in shape))
    return pl.pallas_call(
        body,
        grid=(n // tile,),
        in_specs=[pl.BlockSpec((tile, DIM), lambda i: (i, 0)),
                  pl.BlockSpec((tile, DIM), lambda i: (i, 0)),
                  pl.BlockSpec((tile, DIM), lambda i: (i, 0)),
                  full(DIM, 32), full(32,), full(32, 8), full(8,),
                  full(DIM, DIM), full(DIM, DIM), full(DIM,), full(DIM, 8), full(8,)],
        out_specs=[pl.BlockSpec((tile, 8), lambda i: (i, 0)),
                   pl.BlockSpec((tile, 8), lambda i: (i, 0))],
        out_shape=[jax.ShapeDtypeStruct((n, 8), jnp.float32),
                   jax.ShapeDtypeStruct((n, 8), jnp.float32)],
        interpret=_INTERPRET,
    )(nsel, ssel, tsel, wn1, bn1, wn2p, bn2p, we1s, we1t, be1, we2p, be2p)


def _sc_prologue(node_type_p, embedding, dist_p, sel_p):
    npad = node_type_p.shape[0]
    epad = dist_p.shape[0]
    selpad = sel_p.shape[0]
    half = epad // NC
    rows_t = npad // NW
    n_nch = rows_t // 112
    d_t = half // NS
    sel_t = selpad // NW
    n_sch = sel_t // 128

    mesh = plsc.VectorSubcoreMesh(core_axis_name="c", subcore_axis_name="s",
                                  num_cores=NC, num_subcores=NS)

    @functools.partial(
        pl.kernel,
        out_type=[jax.ShapeDtypeStruct((npad, DIM), jnp.float32),
                  jax.ShapeDtypeStruct((epad,), jnp.float32)],
        mesh=mesh,
        compiler_params=pltpu.CompilerParams(use_tc_tiling_on_sc=False),
        scratch_types=[
            pltpu.VMEM((112,), jnp.int32),
            pltpu.VMEM((112, DIM), jnp.float32),
            pltpu.VMEM((128,), jnp.int32),
            pltpu.VMEM((128,), jnp.int32),
            pltpu.VMEM((128,), jnp.float32),
            pltpu.VMEM_SHARED((half + 8,), jnp.float32),
            pltpu.SemaphoreType.DMA,
        ],
        interpret=_INTERPRET,
    )
    def k(nt_hbm, emb_hbm, dist_hbm, sel_hbm, node_hbm, dm_hbm,
          idx_v, rows_v, sel_v, lsel_v, neg_v, stage, sem):
        c = lax.axis_index("c")
        s = lax.axis_index("s")
        wid = s * NC + c
        cbase = c * half

        pltpu.sync_copy(dist_hbm.at[pl.ds(cbase + s * d_t, d_t)],
                        stage.at[pl.ds(s * d_t, d_t)])
        plsc.subcore_barrier()

        for g in range(8):
            neg_v[pl.ds(g * 16, 16)] = jnp.full((16,), -1.0, jnp.float32)

        def sel_body(j, _):
            pltpu.sync_copy(sel_hbm.at[pl.ds(wid * sel_t + j * 128, 128)], sel_v)
            for g in range(8):
                sv = sel_v[pl.ds(g * 16, 16)]
                inr = (sv >= cbase) & (sv < cbase + half)
                lsel_v[pl.ds(g * 16, 16)] = jnp.where(
                    inr, sv - cbase, jnp.full((16,), half, jnp.int32))
            pltpu.sync_copy(neg_v, stage.at[lsel_v])
            return 0

        lax.fori_loop(0, n_sch, sel_body, 0)
        plsc.subcore_barrier()

        pltpu.sync_copy(stage.at[pl.ds(s * d_t, d_t)],
                        dm_hbm.at[pl.ds(cbase + s * d_t, d_t)])

        def emb_body(j, _):
            base = wid * rows_t + j * 112
            pltpu.sync_copy(nt_hbm.at[pl.ds(base, 112)], idx_v)
            pltpu.async_copy(emb_hbm.at[idx_v], rows_v, sem).wait()
            pltpu.sync_copy(rows_v, node_hbm.at[pl.ds(base, 112)])
            return 0

        lax.fori_loop(0, n_nch, emb_body, 0)

    return k(node_type_p, embedding, dist_p, sel_p)


def _sc_msg(h, src_p, dst_p, new_node):
    epad = h.shape[0]
    npad = new_node.shape[0]
    nhalf = npad // NC
    e_t = epad // NS
    n_ech = e_t // 128
    w_rows = nhalf // NS

    mesh = plsc.VectorSubcoreMesh(core_axis_name="c", subcore_axis_name="s",
                                  num_cores=NC, num_subcores=NS)

    @functools.partial(
        pl.kernel,
        out_type=jax.ShapeDtypeStruct((npad, DIM), jnp.float32),
        mesh=mesh,
        compiler_params=pltpu.CompilerParams(use_tc_tiling_on_sc=False),
        scratch_types=[
            pltpu.VMEM((128,), jnp.int32),
            pltpu.VMEM((128,), jnp.int32),
            pltpu.VMEM((128,), jnp.int32),
            pltpu.VMEM((128, DIM), jnp.float32),
            pltpu.VMEM((128, DIM), jnp.float32),
            pltpu.VMEM((112, DIM), jnp.float32),
            pltpu.VMEM_SHARED((nhalf + 8, DIM), jnp.float32),
            pltpu.SemaphoreType.DMA,
        ],
        interpret=_INTERPRET,
    )
    def k(h_hbm, src_hbm, dst_hbm, nn_hbm, agg_hbm,
          src_v, dst_v, ldst_v, nn_v, h_v, z_v, acc, sem):
        c = lax.axis_index("c")
        s = lax.axis_index("s")
        cbase = c * nhalf

        def zfill(r, _):
            for q in range(DIM // 16):
                z_v[r, pl.ds(q * 16, 16)] = jnp.zeros((16,), jnp.float32)
            return 0

        lax.fori_loop(0, 112, zfill, 0)

        def zero_body(j, _):
            pltpu.sync_copy(z_v, acc.at[pl.ds(s * w_rows + j * 112, 112)])
            return 0

        lax.fori_loop(0, w_rows // 112, zero_body, 0)
        pl.when(s == 0)(lambda: pltpu.sync_copy(
            z_v.at[pl.ds(0, 8)], acc.at[pl.ds(nhalf, 8)]))
        plsc.subcore_barrier()

        def edge_body(j, _):
            base = s * e_t + j * 128
            pltpu.sync_copy(src_hbm.at[pl.ds(base, 128)], src_v)
            gather = pltpu.async_copy(nn_hbm.at[src_v], nn_v, sem)
            pltpu.sync_copy(dst_hbm.at[pl.ds(base, 128)], dst_v)
            for g in range(8):
                dv = dst_v[pl.ds(g * 16, 16)]
                inr = (dv >= cbase) & (dv < cbase + nhalf)
                ldst_v[pl.ds(g * 16, 16)] = jnp.where(
                    inr, dv - cbase, jnp.full((16,), nhalf, jnp.int32))
            pltpu.sync_copy(h_hbm.at[pl.ds(base, 128)], h_v)
            gather.wait()

            def mul_body(e, _):
                for q in range(DIM // 16):
                    h_v[e, pl.ds(q * 16, 16)] = (h_v[e, pl.ds(q * 16, 16)]
                                                 * nn_v[e, pl.ds(q * 16, 16)])
                return 0

            lax.fori_loop(0, 128, mul_body, 0)
            pltpu.sync_copy(h_v, acc.at[ldst_v], add=True)
            return 0

        lax.fori_loop(0, n_ech, edge_body, 0)
        plsc.subcore_barrier()

        pltpu.sync_copy(acc.at[pl.ds(s * w_rows, w_rows)],
                        agg_hbm.at[pl.ds(cbase + s * w_rows, w_rows)])

    return k(h, src_p, dst_p, new_node)


def _sc_gather3(feature, ni_p, si_p, ti_p):
    nsel = ni_p.shape[0]
    per_w = nsel // NW
    n_ch = per_w // 128

    mesh = plsc.VectorSubcoreMesh(core_axis_name="c", subcore_axis_name="s",
                                  num_cores=NC, num_subcores=NS)

    @functools.partial(
        pl.kernel,
        out_type=[jax.ShapeDtypeStruct((nsel, DIM), jnp.float32)] * 3,
        mesh=mesh,
        compiler_params=pltpu.CompilerParams(use_tc_tiling_on_sc=False),
        scratch_types=[
            pltpu.VMEM((128,), jnp.int32),
            pltpu.VMEM((128, DIM), jnp.float32),
            pltpu.SemaphoreType.DMA,
        ],
        interpret=_INTERPRET,
    )
    def k(f_hbm, ni_hbm, si_hbm, ti_hbm, no_hbm, so_hbm, to_hbm, idx_v, rows_v, sem):
        c = lax.axis_index("c")
        s = lax.axis_index("s")
        wid = s * NC + c

        def gather_one(idx_hbm, out_hbm):
            def body(j, _):
                base = wid * per_w + j * 128
                pltpu.sync_copy(idx_hbm.at[pl.ds(base, 128)], idx_v)
                pltpu.async_copy(f_hbm.at[idx_v], rows_v, sem).wait()
                pltpu.sync_copy(rows_v, out_hbm.at[pl.ds(base, 128)])
                return 0
            lax.fori_loop(0, n_ch, body, 0)

        gather_one(ni_hbm, no_hbm)
        gather_one(si_hbm, so_hbm)
        gather_one(ti_hbm, to_hbm)

    return k(feature, ni_p, si_p, ti_p)


def kernel(node_type, edge_index, distance, node_index, source_index, target_index,
           select_edge_index, embedding, edge_mask, conv_params,
           W_nt1, b_nt1, W_nt2, b_nt2, W_et1, b_et1, W_et2, b_et2):
    n = node_type.shape[0]
    e = distance.shape[0]
    nsel = node_index.shape[0]
    esel = select_edge_index.shape[0]

    def rup(x, m):
        return ((x + m - 1) // m) * m

    npad = rup(n, NW * 112)
    epad = rup(e, NW * 128)
    selpad = rup(esel, NW * 128)
    nselpad = rup(nsel, NW * 128)

    nt_p = jnp.pad(node_type, (0, npad - n))
    src_p = jnp.pad(edge_index[0], (0, epad - e))
    dst_p = jnp.pad(edge_index[1], (0, epad - e), constant_values=npad)
    dist_p = jnp.pad(distance, (0, epad - e))
    sel_p = jnp.pad(select_edge_index, (0, selpad - esel), constant_values=epad)
    ni_p = jnp.pad(node_index, (0, nselpad - nsel))
    si_p = jnp.pad(source_index, (0, nselpad - nsel))
    ti_p = jnp.pad(target_index, (0, nselpad - nsel))

    node, dist_m = _sc_prologue(nt_p, embedding, dist_p, sel_p)

    for p in conv_params:
        nn = _tc_matmul(node, p["W_nl1"])
        h = _tc_h(dist_m, edge_mask, p["W_cf1"], p["b_cf1"], p["W_cf2"], p["b_cf2"])
        agg = _sc_msg(h, src_p, dst_p, nn)
        node = _tc_update(node, agg, p["W_nl2"], p["b_nl2"], p["W_nl3"], p["b_nl3"])

    nrows, srows, trows = _sc_gather3(node, ni_p, si_p, ti_p)

    wn2p = jnp.pad(W_nt2, ((0, 0), (0, 8 - W_nt2.shape[1])))
    bn2p = jnp.pad(b_nt2, (0, 8 - b_nt2.shape[0]))
    we2p = jnp.pad(W_et2, ((0, 0), (0, 8 - W_et2.shape[1])))
    be2p = jnp.pad(b_et2, (0, 8 - b_et2.shape[0]))
    nt8, et8 = _tc_heads(nrows, srows, trows, W_nt1, b_nt1, wn2p, bn2p,
                         W_et1[:DIM], W_et1[DIM:], b_et1, we2p, be2p)
    return nt8[:nsel, :3], et8[:nsel, :5]


# speedup vs baseline: 3.7466x; 1.1857x over previous
import functools

import jax
import jax.numpy as jnp
from jax import lax
from jax.experimental import pallas as pl
from jax.experimental.pallas import tpu as pltpu
from jax.experimental.pallas import tpu_sc as plsc

DIM = 64
CUTOFF = 5.0
N_CENTERS = 50
L = 16
NC = 2
NS = 16
NW = NC * NS

_INTERPRET = False


def _softplus(x, beta=0.5, threshold=14.0):
    return jnp.where(beta * x > threshold, x,
                     (1.0 / beta) * jnp.log1p(jnp.exp(jnp.minimum(beta * x, threshold))))


def _tc_h(dist_m, edge_mask, w1, b1, w2, b2, tile=1024):
    epad = dist_m.shape[0]
    gap = CUTOFF / (N_CENTERS - 1)

    def body(d_ref, em_ref, w1_ref, b1_ref, w2_ref, b2_ref, o_ref):
        d2 = d_ref[...][:, None]
        centers = lax.broadcasted_iota(
            jnp.int32, (1, N_CENTERS), 1).astype(jnp.float32) * gap
        rbf = jnp.exp((-1.0 / gap) * (d2 - centers) ** 2)
        rbf = jnp.where(d2 < 0.0, em_ref[...][None, :], rbf)
        hh = _softplus(jnp.dot(rbf, w1_ref[...], preferred_element_type=jnp.float32)
                       + b1_ref[...][None, :])
        o_ref[...] = (jnp.dot(hh, w2_ref[...], preferred_element_type=jnp.float32)
                      + b2_ref[...][None, :])

    return pl.pallas_call(
        body,
        grid=(epad // tile,),
        in_specs=[
            pl.BlockSpec((tile,), lambda i: (i,)),
            pl.BlockSpec((N_CENTERS,), lambda i: (0,)),
            pl.BlockSpec((N_CENTERS, DIM), lambda i: (0, 0)),
            pl.BlockSpec((DIM,), lambda i: (0,)),
            pl.BlockSpec((DIM, DIM), lambda i: (0, 0)),
            pl.BlockSpec((DIM,), lambda i: (0,)),
        ],
        out_specs=pl.BlockSpec((tile, DIM), lambda i: (i, 0)),
        out_shape=jax.ShapeDtypeStruct((epad, DIM), jnp.float32),
        interpret=_INTERPRET,
    )(dist_m, edge_mask, w1, b1, w2, b2)


def _tc_matmul(x, w, tile=1024):
    n = x.shape[0]

    def body(x_ref, w_ref, o_ref):
        o_ref[...] = jnp.dot(x_ref[...], w_ref[...], preferred_element_type=jnp.float32)

    return pl.pallas_call(
        body,
        grid=(n // tile,),
        in_specs=[pl.BlockSpec((tile, DIM), lambda i: (i, 0)),
                  pl.BlockSpec((DIM, DIM), lambda i: (0, 0))],
        out_specs=pl.BlockSpec((tile, DIM), lambda i: (i, 0)),
        out_shape=jax.ShapeDtypeStruct((n, DIM), jnp.float32),
        interpret=_INTERPRET,
    )(x, w)


def _tc_update(node, agg, w2, b2, w3, b3, tile=1024):
    n = node.shape[0]

    def body(n_ref, a_ref, w2_ref, b2_ref, w3_ref, b3_ref, o_ref):
        cf = _softplus(jnp.dot(a_ref[...], w2_ref[...], preferred_element_type=jnp.float32)
                       + b2_ref[...][None, :])
        o_ref[...] = n_ref[...] + (
            jnp.dot(cf, w3_ref[...], preferred_element_type=jnp.float32)
            + b3_ref[...][None, :])

    return pl.pallas_call(
        body,
        grid=(n // tile,),
        in_specs=[pl.BlockSpec((tile, DIM), lambda i: (i, 0)),
                  pl.BlockSpec((tile, DIM), lambda i: (i, 0)),
                  pl.BlockSpec((DIM, DIM), lambda i: (0, 0)),
                  pl.BlockSpec((DIM,), lambda i: (0,)),
                  pl.BlockSpec((DIM, DIM), lambda i: (0, 0)),
                  pl.BlockSpec((DIM,), lambda i: (0,))],
        out_specs=pl.BlockSpec((tile, DIM), lambda i: (i, 0)),
        out_shape=jax.ShapeDtypeStruct((n, DIM), jnp.float32),
        interpret=_INTERPRET,
    )(node, agg, w2, b2, w3, b3)


def _tc_heads(nsel, ssel, tsel, wn1, bn1, wn2p, bn2p, we1s, we1t, be1, we2p, be2p,
              tile=512):
    n = nsel.shape[0]

    def body(ns_ref, ss_ref, ts_ref, wn1_ref, bn1_ref, wn2_ref, bn2_ref,
             we1s_ref, we1t_ref, be1_ref, we2_ref, be2_ref, nt_ref, et_ref):
        f32 = jnp.float32
        t1 = jnp.dot(ns_ref[...], wn1_ref[...], preferred_element_type=f32) + bn1_ref[...][None, :]
        nt_ref[...] = jnp.dot(t1, wn2_ref[...], preferred_element_type=f32) + bn2_ref[...][None, :]
        e1 = (jnp.dot(ss_ref[...], we1s_ref[...], preferred_element_type=f32)
              + jnp.dot(ts_ref[...], we1t_ref[...], preferred_element_type=f32)
              + be1_ref[...][None, :])
        et_ref[...] = jnp.dot(e1, we2_ref[...], preferred_element_type=f32) + be2_ref[...][None, :]

    full = lambda *shape: pl.BlockSpec(shape, lambda i: tuple(0 for _ in shape))
    return pl.pallas_call(
        body,
        grid=(n // tile,),
        in_specs=[pl.BlockSpec((tile, DIM), lambda i: (i, 0)),
                  pl.BlockSpec((tile, DIM), lambda i: (i, 0)),
                  pl.BlockSpec((tile, DIM), lambda i: (i, 0)),
                  full(DIM, 32), full(32,), full(32, 8), full(8,),
                  full(DIM, DIM), full(DIM, DIM), full(DIM,), full(DIM, 8), full(8,)],
        out_specs=[pl.BlockSpec((tile, 8), lambda i: (i, 0)),
                   pl.BlockSpec((tile, 8), lambda i: (i, 0))],
        out_shape=[jax.ShapeDtypeStruct((n, 8), jnp.float32),
                   jax.ShapeDtypeStruct((n, 8), jnp.float32)],
        interpret=_INTERPRET,
    )(nsel, ssel, tsel, wn1, bn1, wn2p, bn2p, we1s, we1t, be1, we2p, be2p)


def _sc_prologue(node_type_p, embedding, dist_p, sel_p):
    npad = node_type_p.shape[0]
    epad = dist_p.shape[0]
    selpad = sel_p.shape[0]
    half = epad // NC
    rows_t = npad // NW
    n_nch = rows_t // 112
    d_t = half // NS
    sel_t = selpad // NW
    n_sch = sel_t // 128

    mesh = plsc.VectorSubcoreMesh(core_axis_name="c", subcore_axis_name="s",
                                  num_cores=NC, num_subcores=NS)

    @functools.partial(
        pl.kernel,
        out_type=[jax.ShapeDtypeStruct((npad, DIM), jnp.float32),
                  jax.ShapeDtypeStruct((epad,), jnp.float32)],
        mesh=mesh,
        compiler_params=pltpu.CompilerParams(use_tc_tiling_on_sc=False),
        scratch_types=[
            pltpu.VMEM((112,), jnp.int32),
            pltpu.VMEM((112, DIM), jnp.float32),
            pltpu.VMEM((128,), jnp.int32),
            pltpu.VMEM((128,), jnp.int32),
            pltpu.VMEM((128,), jnp.float32),
            pltpu.VMEM_SHARED((half + 8,), jnp.float32),
            pltpu.SemaphoreType.DMA,
        ],
        interpret=_INTERPRET,
    )
    def k(nt_hbm, emb_hbm, dist_hbm, sel_hbm, node_hbm, dm_hbm,
          idx_v, rows_v, sel_v, lsel_v, neg_v, stage, sem):
        c = lax.axis_index("c")
        s = lax.axis_index("s")
        wid = s * NC + c
        cbase = c * half

        pltpu.sync_copy(dist_hbm.at[pl.ds(cbase + s * d_t, d_t)],
                        stage.at[pl.ds(s * d_t, d_t)])
        plsc.subcore_barrier()

        for g in range(8):
            neg_v[pl.ds(g * 16, 16)] = jnp.full((16,), -1.0, jnp.float32)

        def sel_body(j, _):
            pltpu.sync_copy(sel_hbm.at[pl.ds(wid * sel_t + j * 128, 128)], sel_v)
            for g in range(8):
                sv = sel_v[pl.ds(g * 16, 16)]
                inr = (sv >= cbase) & (sv < cbase + half)
                lsel_v[pl.ds(g * 16, 16)] = jnp.where(
                    inr, sv - cbase, jnp.full((16,), half, jnp.int32))
            pltpu.sync_copy(neg_v, stage.at[lsel_v])
            return 0

        lax.fori_loop(0, n_sch, sel_body, 0)
        plsc.subcore_barrier()

        pltpu.sync_copy(stage.at[pl.ds(s * d_t, d_t)],
                        dm_hbm.at[pl.ds(cbase + s * d_t, d_t)])

        def emb_body(j, _):
            base = wid * rows_t + j * 112
            pltpu.sync_copy(nt_hbm.at[pl.ds(base, 112)], idx_v)
            pltpu.async_copy(emb_hbm.at[idx_v], rows_v, sem).wait()
            pltpu.sync_copy(rows_v, node_hbm.at[pl.ds(base, 112)])
            return 0

        lax.fori_loop(0, n_nch, emb_body, 0)

    return k(node_type_p, embedding, dist_p, sel_p)


def _sc_msg(h, src_p, dst_p, new_node):
    epad = h.shape[0]
    npad = new_node.shape[0]
    nhalf = npad // NC
    e_t = epad // NS
    ch = 128
    n_ech = e_t // ch
    w_rows = nhalf // NS

    mesh = plsc.VectorSubcoreMesh(core_axis_name="c", subcore_axis_name="s",
                                  num_cores=NC, num_subcores=NS)

    @functools.partial(
        pl.kernel,
        out_type=jax.ShapeDtypeStruct((npad, DIM), jnp.float32),
        mesh=mesh,
        compiler_params=pltpu.CompilerParams(use_tc_tiling_on_sc=False),
        scratch_types=(
            [pltpu.VMEM((ch,), jnp.int32)] * 2
            + [pltpu.VMEM((ch,), jnp.int32)] * 2
            + [pltpu.VMEM((ch,), jnp.int32)] * 2
            + [pltpu.VMEM((ch, DIM), jnp.float32)] * 2
            + [pltpu.VMEM((ch, DIM), jnp.float32)]
            + [pltpu.SemaphoreType.DMA] * 7
            + [pltpu.VMEM_SHARED((nhalf + 8, DIM), jnp.float32)]
        ),
        interpret=_INTERPRET,
    )
    def k(h_hbm, src_hbm, dst_hbm, nn_hbm, agg_hbm,
          src0, src1, dst0, dst1, ld0, ld1, nn0, nn1, hv,
          ls0, ls1, hs, gs0, gs1, ss0, ss1, acc):
        src_v = [src0, src1]
        dst_v = [dst0, dst1]
        ldst_v = [ld0, ld1]
        nn_v = [nn0, nn1]
        lsem = [ls0, ls1]
        gsem = [gs0, gs1]
        ssem = [ss0, ss1]
        c = lax.axis_index("c")
        s = lax.axis_index("s")
        cbase = c * nhalf

        def zfill(r, _):
            for q in range(DIM // 16):
                hv[r, pl.ds(q * 16, 16)] = jnp.zeros((16,), jnp.float32)
            return 0

        lax.fori_loop(0, ch, zfill, 0)

        def zero_body(j, _):
            pltpu.sync_copy(hv, acc.at[pl.ds(s * w_rows + j * ch, ch)])
            return 0

        lax.fori_loop(0, w_rows // ch, zero_body, 0)
        rem = w_rows % ch
        if rem:
            pltpu.sync_copy(hv.at[pl.ds(0, rem)],
                            acc.at[pl.ds(s * w_rows + (w_rows // ch) * ch, rem)])
        pl.when(s == 0)(lambda: pltpu.sync_copy(
            hv.at[pl.ds(0, 8)], acc.at[pl.ds(nhalf, 8)]))
        plsc.subcore_barrier()

        def ebase(j):
            return s * e_t + j * ch

        def issue_sd(j, b):
            pltpu.async_copy(src_hbm.at[pl.ds(ebase(j), ch)], src_v[b], lsem[b])
            pltpu.async_copy(dst_hbm.at[pl.ds(ebase(j), ch)], dst_v[b], lsem[b])

        def wait_sd(j, b):
            pltpu.make_async_copy(src_hbm.at[pl.ds(ebase(j), ch)], src_v[b], lsem[b]).wait()
            pltpu.make_async_copy(dst_hbm.at[pl.ds(ebase(j), ch)], dst_v[b], lsem[b]).wait()

        def issue_h(j):
            pltpu.async_copy(h_hbm.at[pl.ds(ebase(j), ch)], hv, hs)

        def wait_h(j):
            pltpu.make_async_copy(h_hbm.at[pl.ds(ebase(j), ch)], hv, hs).wait()

        def issue_gather(b):
            pltpu.async_copy(nn_hbm.at[src_v[b]], nn_v[b], gsem[b])

        def wait_gather(b):
            pltpu.make_async_copy(nn_hbm.at[src_v[b]], nn_v[b], gsem[b]).wait()

        def issue_scatter(b):
            pltpu.async_copy(nn_v[b], acc.at[ldst_v[b]], ssem[b], add=True)

        def wait_scatter(b):
            pltpu.make_async_copy(nn_v[b], acc.at[ldst_v[b]], ssem[b]).wait()

        issue_sd(0, 0)
        issue_h(0)
        wait_sd(0, 0)
        issue_gather(0)

        def edge_body(jj, _):
            for b in range(2):
                j = jj * 2 + b
                ob = 1 - b

                pl.when(j + 1 < n_ech)(lambda: issue_sd(j + 1, ob))

                for g in range(ch // 16):
                    dv = dst_v[b][pl.ds(g * 16, 16)]
                    inr = (dv >= cbase) & (dv < cbase + nhalf)
                    ldst_v[b][pl.ds(g * 16, 16)] = jnp.where(
                        inr, dv - cbase, jnp.full((16,), nhalf, jnp.int32))
                wait_h(j)
                wait_gather(b)

                def mul_body(m, _):
                    for ee in range(4):
                        e = m * 4 + ee
                        for q in range(DIM // 16):
                            nn_v[b][e, pl.ds(q * 16, 16)] = (
                                nn_v[b][e, pl.ds(q * 16, 16)]
                                * hv[e, pl.ds(q * 16, 16)])
                    return 0

                lax.fori_loop(0, ch // 4, mul_body, 0)
                issue_scatter(b)

                def finish_next():
                    issue_h(j + 1)
                    wait_sd(j + 1, ob)
                    pl.when(j >= 1)(lambda: wait_scatter(ob))
                    issue_gather(ob)

                pl.when(j + 1 < n_ech)(finish_next)
            return 0

        lax.fori_loop(0, n_ech // 2, edge_body, 0)
        wait_scatter(0)
        wait_scatter(1)
        plsc.subcore_barrier()

        pltpu.sync_copy(acc.at[pl.ds(s * w_rows, w_rows)],
                        agg_hbm.at[pl.ds(cbase + s * w_rows, w_rows)])

    return k(h, src_p, dst_p, new_node)


def _sc_gather3(feature, ni_p, si_p, ti_p):
    nsel = ni_p.shape[0]
    per_w = nsel // NW
    n_ch = per_w // 128

    mesh = plsc.VectorSubcoreMesh(core_axis_name="c", subcore_axis_name="s",
                                  num_cores=NC, num_subcores=NS)

    @functools.partial(
        pl.kernel,
        out_type=[jax.ShapeDtypeStruct((nsel, DIM), jnp.float32)] * 3,
        mesh=mesh,
        compiler_params=pltpu.CompilerParams(use_tc_tiling_on_sc=False),
        scratch_types=[
            pltpu.VMEM((128,), jnp.int32),
            pltpu.VMEM((128, DIM), jnp.float32),
            pltpu.SemaphoreType.DMA,
        ],
        interpret=_INTERPRET,
    )
    def k(f_hbm, ni_hbm, si_hbm, ti_hbm, no_hbm, so_hbm, to_hbm, idx_v, rows_v, sem):
        c = lax.axis_index("c")
        s = lax.axis_index("s")
        wid = s * NC + c

        def gather_one(idx_hbm, out_hbm):
            def body(j, _):
                base = wid * per_w + j * 128
                pltpu.sync_copy(idx_hbm.at[pl.ds(base, 128)], idx_v)
                pltpu.async_copy(f_hbm.at[idx_v], rows_v, sem).wait()
                pltpu.sync_copy(rows_v, out_hbm.at[pl.ds(base, 128)])
                return 0
            lax.fori_loop(0, n_ch, body, 0)

        gather_one(ni_hbm, no_hbm)
        gather_one(si_hbm, so_hbm)
        gather_one(ti_hbm, to_hbm)

    return k(feature, ni_p, si_p, ti_p)


def kernel(node_type, edge_index, distance, node_index, source_index, target_index,
           select_edge_index, embedding, edge_mask, conv_params,
           W_nt1, b_nt1, W_nt2, b_nt2, W_et1, b_et1, W_et2, b_et2):
    n = node_type.shape[0]
    e = distance.shape[0]
    nsel = node_index.shape[0]
    esel = select_edge_index.shape[0]

    def rup(x, m):
        return ((x + m - 1) // m) * m

    npad = rup(n, NW * 112)
    epad = rup(e, NS * 256)
    selpad = rup(esel, NW * 128)
    nselpad = rup(nsel, NW * 128)

    nt_p = jnp.pad(node_type, (0, npad - n))
    src_p = jnp.pad(edge_index[0], (0, epad - e))
    dst_p = jnp.pad(edge_index[1], (0, epad - e), constant_values=npad)
    dist_p = jnp.pad(distance, (0, epad - e))
    sel_p = jnp.pad(select_edge_index, (0, selpad - esel), constant_values=epad)
    ni_p = jnp.pad(node_index, (0, nselpad - nsel))
    si_p = jnp.pad(source_index, (0, nselpad - nsel))
    ti_p = jnp.pad(target_index, (0, nselpad - nsel))

    node, dist_m = _sc_prologue(nt_p, embedding, dist_p, sel_p)

    for p in conv_params:
        nn = _tc_matmul(node, p["W_nl1"])
        h = _tc_h(dist_m, edge_mask, p["W_cf1"], p["b_cf1"], p["W_cf2"], p["b_cf2"])
        agg = _sc_msg(h, src_p, dst_p, nn)
        node = _tc_update(node, agg, p["W_nl2"], p["b_nl2"], p["W_nl3"], p["b_nl3"])

    nrows, srows, trows = _sc_gather3(node, ni_p, si_p, ti_p)

    wn2p = jnp.pad(W_nt2, ((0, 0), (0, 8 - W_nt2.shape[1])))
    bn2p = jnp.pad(b_nt2, (0, 8 - b_nt2.shape[0]))
    we2p = jnp.pad(W_et2, ((0, 0), (0, 8 - W_et2.shape[1])))
    be2p = jnp.pad(b_et2, (0, 8 - b_et2.shape[0]))
    nt8, et8 = _tc_heads(nrows, srows, trows, W_nt1, b_nt1, wn2p, bn2p,
                         W_et1[:DIM], W_et1[DIM:], b_et1, we2p, be2p)
    return nt8[:nsel, :3], et8[:nsel, :5]
